# baseline (speedup 1.0000x reference)
import functools

import jax
import jax.numpy as jnp
from jax import lax
from jax.experimental import pallas as pl
from jax.experimental.pallas import tpu as pltpu
from jax.experimental.pallas import tpu_sc as plsc

D_MODEL = 64
SCALE = 8.0
CHUNK = 128
NBUF = 2
LANES = 16


def _body(idx_hbm, table_hbm, out_hbm, idx_v, rows0, rows1, gsem0, gsem1):
    nc = 2
    wid = lax.axis_index("s") * nc + lax.axis_index("c")
    n_chunks = idx_hbm.shape[1]
    bpw = n_chunks * CHUNK
    base = wid * bpw

    rows = (rows0, rows1)
    gsem = (gsem0, gsem1)

    pltpu.sync_copy(idx_hbm.at[wid], idx_v)

    def gather(g, b):
        return pltpu.make_async_copy(table_hbm.at[idx_v.at[g]], rows[b], gsem[b])

    for b in range(NBUF):
        gather(b, b).start()

    def scale_rows(b):
        def srow(r, carry):
            for l in range(D_MODEL // LANES):
                sl = (r, pl.ds(l * LANES, LANES))
                rows[b][sl] = rows[b][sl] * SCALE
            return carry
        lax.fori_loop(0, CHUNK, srow, 0)

    def group(i, carry):
        for b in range(NBUF):
            g = i * NBUF + b
            gather(g, b).wait()
            scale_rows(b)
            pltpu.sync_copy(rows[b], out_hbm.at[pl.ds(base + g * CHUNK, CHUNK)])
            gather(g + NBUF, b).start()
        return carry

    n_groups = n_chunks // NBUF
    lax.fori_loop(0, n_groups - 1, group, 0)

    for b in range(NBUF):
        g = (n_groups - 1) * NBUF + b
        gather(g, b).wait()
        scale_rows(b)
        pltpu.sync_copy(rows[b], out_hbm.at[pl.ds(base + g * CHUNK, CHUNK)])


def kernel(x, embed_weight):
    orig_shape = x.shape
    idx = x.astype(jnp.int32).reshape(-1)
    n = idx.shape[0]

    info = plsc.get_sparse_core_info()
    nw = info.num_cores * info.num_subcores
    bpw = n // nw
    n_chunks = bpw // CHUNK
    idx3 = idx.reshape(nw, n_chunks, CHUNK)

    mesh = plsc.VectorSubcoreMesh(core_axis_name="c", subcore_axis_name="s")
    run = pl.kernel(
        _body,
        out_type=jax.ShapeDtypeStruct((n, D_MODEL), jnp.float32),
        mesh=mesh,
        scratch_types=[
            pltpu.VMEM((n_chunks, CHUNK), jnp.int32),
            pltpu.VMEM((CHUNK, D_MODEL), jnp.float32),
            pltpu.VMEM((CHUNK, D_MODEL), jnp.float32),
            pltpu.SemaphoreType.DMA,
            pltpu.SemaphoreType.DMA,
        ],
        compiler_params=pltpu.CompilerParams(use_tc_tiling_on_sc=False),
    )
    out = run(idx3, embed_weight)
    return out.reshape(*orig_shape, D_MODEL)


# speedup vs baseline: 1.0581x; 1.0581x over previous
import jax
import jax.numpy as jnp
from jax import lax
from jax.experimental import pallas as pl
from jax.experimental.pallas import tpu as pltpu
from jax.experimental.pallas import tpu_sc as plsc

D_MODEL = 64
SCALE = 8.0
CHUNK = 128
NBUF = 4
LANES = 16


def _body(idx_hbm, table_hbm, out_hbm, idx_v, gbufs, obufs, gsems, osems):
    nc = 2
    wid = lax.axis_index("s") * nc + lax.axis_index("c")
    n_chunks = idx_hbm.shape[1]
    bpw = n_chunks * CHUNK
    base = wid * bpw

    pltpu.sync_copy(idx_hbm.at[wid], idx_v)

    def gather(g, b):
        return pltpu.make_async_copy(table_hbm.at[idx_v.at[g]], gbufs[b], gsems[b])

    def copy_out(g, b):
        return pltpu.make_async_copy(
            obufs[b], out_hbm.at[pl.ds(base + g * CHUNK, CHUNK)], osems[b])

    def scale(b):
        @plsc.parallel_loop(0, CHUNK, unroll=4)
        def _(r):
            for l in range(D_MODEL // LANES):
                sl = (r, pl.ds(l * LANES, LANES))
                obufs[b][sl] = gbufs[b][sl] * SCALE

    for b in range(NBUF):
        gather(b, b).start()

    for b in range(NBUF):
        gather(b, b).wait()
        scale(b)
        copy_out(b, b).start()
        gather(b + NBUF, b).start()

    def group(i, carry):
        for b in range(NBUF):
            g = i * NBUF + b
            gather(g, b).wait()
            copy_out(g - NBUF, b).wait()
            scale(b)
            copy_out(g, b).start()
            gather(g + NBUF, b).start()
        return carry

    n_groups = n_chunks // NBUF
    lax.fori_loop(1, n_groups - 1, group, 0)

    for b in range(NBUF):
        g = (n_groups - 1) * NBUF + b
        gather(g, b).wait()
        copy_out(g - NBUF, b).wait()
        scale(b)
        copy_out(g, b).start()
    for b in range(NBUF):
        copy_out((n_groups - 1) * NBUF + b, b).wait()


def kernel(x, embed_weight):
    orig_shape = x.shape
    idx = x.astype(jnp.int32).reshape(-1)
    n = idx.shape[0]

    info = plsc.get_sparse_core_info()
    nw = info.num_cores * info.num_subcores
    bpw = n // nw
    n_chunks = bpw // CHUNK
    idx3 = idx.reshape(nw, n_chunks, CHUNK)

    mesh = plsc.VectorSubcoreMesh(core_axis_name="c", subcore_axis_name="s")

    def body(idx_hbm, table_hbm, out_hbm, idx_v, *scratch):
        gbufs = scratch[0:NBUF]
        obufs = scratch[NBUF:2 * NBUF]
        gsems = scratch[2 * NBUF:3 * NBUF]
        osems = scratch[3 * NBUF:4 * NBUF]
        _body(idx_hbm, table_hbm, out_hbm, idx_v, gbufs, obufs, gsems, osems)

    run = pl.kernel(
        body,
        out_type=jax.ShapeDtypeStruct((n, D_MODEL), jnp.float32),
        mesh=mesh,
        scratch_types=(
            [pltpu.VMEM((n_chunks, CHUNK), jnp.int32)]
            + [pltpu.VMEM((CHUNK, D_MODEL), jnp.float32) for _ in range(2 * NBUF)]
            + [pltpu.SemaphoreType.DMA for _ in range(2 * NBUF)]
        ),
        compiler_params=pltpu.CompilerParams(use_tc_tiling_on_sc=False),
    )
    out = run(idx3, embed_weight)
    return out.reshape(*orig_shape, D_MODEL)


# speedup vs baseline: 1.0597x; 1.0015x over previous
import jax
import jax.numpy as jnp
from jax import lax
from jax.experimental import pallas as pl
from jax.experimental.pallas import tpu as pltpu
from jax.experimental.pallas import tpu_sc as plsc

D_MODEL = 64
SCALE = 8.0
SZ = (128, 72)
OFF = (0, 128)
LANES = 16
NW = 32


def _body(x_hbm, table_hbm, out_hbm, idx_v, gbufs, obufs, gsems, osems):
    nc = 2
    wid = lax.axis_index("s") * nc + lax.axis_index("c")
    n_rows = x_hbm.shape[0] // NW
    row0 = wid * n_rows

    pltpu.sync_copy(x_hbm.at[pl.ds(row0, n_rows)], idx_v)

    def gather(row, h, b):
        src = table_hbm.at[idx_v.at[row, pl.ds(OFF[h], SZ[h])]]
        dst = gbufs[b] if h == 0 else gbufs[b].at[pl.ds(0, SZ[1])]
        return pltpu.make_async_copy(src, dst, gsems[b])

    def copy_out(row, h, b):
        src = obufs[b] if h == 0 else obufs[b].at[pl.ds(0, SZ[1])]
        dst = out_hbm.at[row0 + row, pl.ds(OFF[h], SZ[h])]
        return pltpu.make_async_copy(src, dst, osems[b])

    def scale(h, b):
        @plsc.parallel_loop(0, SZ[h], unroll=4)
        def _(r):
            for l in range(D_MODEL // LANES):
                sl = (r, pl.ds(l * LANES, LANES))
                obufs[b][sl] = gbufs[b][sl] * SCALE

    for rr in range(2):
        for h in range(2):
            gather(rr, h, 2 * rr + h).start()

    for rr in range(2):
        for h in range(2):
            b = 2 * rr + h
            gather(rr, h, b).wait()
            scale(h, b)
            copy_out(rr, h, b).start()
            gather(2 + rr, h, b).start()

    def group(i, carry):
        for rr in range(2):
            row = 2 * i + rr
            for h in range(2):
                b = 2 * rr + h
                gather(row, h, b).wait()
                copy_out(row - 2, h, b).wait()
                scale(h, b)
                copy_out(row, h, b).start()
                gather(row + 2, h, b).start()
        return carry

    n_groups = n_rows // 2
    lax.fori_loop(1, n_groups - 1, group, 0)

    for rr in range(2):
        row = n_rows - 2 + rr
        for h in range(2):
            b = 2 * rr + h
            gather(row, h, b).wait()
            copy_out(row - 2, h, b).wait()
            scale(h, b)
            copy_out(row, h, b).start()
    for rr in range(2):
        for h in range(2):
            copy_out(n_rows - 2 + rr, h, 2 * rr + h).wait()


def kernel(x, embed_weight):
    n_x_rows, row_len = x.shape
    xi = x.astype(jnp.int32)

    mesh = plsc.VectorSubcoreMesh(core_axis_name="c", subcore_axis_name="s")
    n_rows = n_x_rows // NW

    def body(x_hbm, table_hbm, out_hbm, idx_v, *scratch):
        gbufs = scratch[0:4]
        obufs = scratch[4:8]
        gsems = scratch[8:12]
        osems = scratch[12:16]
        _body(x_hbm, table_hbm, out_hbm, idx_v, gbufs, obufs, gsems, osems)

    run = pl.kernel(
        body,
        out_type=jax.ShapeDtypeStruct((n_x_rows, row_len, D_MODEL), jnp.float32),
        mesh=mesh,
        scratch_types=(
            [pltpu.VMEM((n_rows, row_len), jnp.int32)]
            + [pltpu.VMEM((SZ[0], D_MODEL), jnp.float32) for _ in range(8)]
            + [pltpu.SemaphoreType.DMA for _ in range(8)]
        ),
        compiler_params=pltpu.CompilerParams(use_tc_tiling_on_sc=False),
    )
    return run(xi, embed_weight)
